# baseline (speedup 1.0000x reference)
import functools
import math

import jax
import jax.numpy as jnp
from jax import lax
from jax.experimental import pallas as pl
from jax.experimental.pallas import tpu as pltpu
from jax.experimental.pallas import tpu_sc as plsc

D_MODEL = 64
VOCAB = 1000000
BATCH = 4096
SEQ = 200
SCALE = math.sqrt(D_MODEL)

_INFO = plsc.get_sparse_core_info()
_NC, _NS, _L = _INFO.num_cores, _INFO.num_subcores, _INFO.num_lanes
_NW = _NC * _NS

_B = BATCH * SEQ
_B_PER_W = _B // _NW
_BATCH_PER_W = BATCH // _NW
_ROWS_PER_CHUNK = 2
_CHUNK = _ROWS_PER_CHUNK * SEQ
_N_CHUNKS = _B_PER_W // _CHUNK
_IDX_SUB = 80
_N_SUB = _CHUNK // _IDX_SUB
_ROWS_UNROLL = 4


def _emb_kernel(x_hbm, table_hbm, out_hbm, idx_all, rows, sem_g0, sem_g1,
                sem_o0, sem_o1):
    wid = lax.axis_index("s") * _NC + lax.axis_index("c")
    base = wid * _B_PER_W
    brow0 = wid * _BATCH_PER_W
    sem_g = (sem_g0, sem_g1)
    sem_o = (sem_o0, sem_o1)

    pltpu.sync_copy(x_hbm.at[pl.ds(pl.multiple_of(base, 8), _B_PER_W)], idx_all)

    def fire_gathers(g, b):
        for k in range(_N_SUB):
            off = pl.multiple_of(g * _CHUNK + k * _IDX_SUB, 8)
            pltpu.async_copy(
                table_hbm.at[idx_all.at[pl.ds(off, _IDX_SUB)]],
                rows.at[b, pl.ds(k * _IDX_SUB, _IDX_SUB)],
                sem_g[b],
            )

    def wait_gathers(b):
        pltpu.make_async_copy(
            table_hbm.at[pl.ds(0, _CHUNK)], rows.at[b], sem_g[b]
        ).wait()

    def fire_writeout(g, b):
        br = brow0 + g * _ROWS_PER_CHUNK
        for j in range(_ROWS_PER_CHUNK):
            pltpu.async_copy(
                rows.at[b, pl.ds(j * SEQ, SEQ)], out_hbm.at[br + j], sem_o[b]
            )

    def wait_writeout(b):
        for j in range(_ROWS_PER_CHUNK):
            pltpu.make_async_copy(
                rows.at[b, pl.ds(j * SEQ, SEQ)], out_hbm.at[0], sem_o[b]
            ).wait()

    def scale_chunk(b):
        rr = rows.at[b]

        @plsc.parallel_loop(0, _CHUNK // _ROWS_UNROLL, unroll=2)
        def scale_body(r4):
            r0 = r4 * _ROWS_UNROLL
            for dr in range(_ROWS_UNROLL):
                for c4 in range(D_MODEL // _L):
                    sl = pl.ds(c4 * _L, _L)
                    rr[r0 + dr, sl] = rr[r0 + dr, sl] * SCALE

    fire_gathers(0, 0)

    def pair_body(gg, _):
        for b in (0, 1):
            g = gg * 2 + b

            @pl.when(g >= 1)
            def _():
                wait_writeout(1 - b)

            @pl.when(g + 1 < _N_CHUNKS)
            def _():
                fire_gathers(g + 1, 1 - b)

            wait_gathers(b)
            scale_chunk(b)
            fire_writeout(g, b)
        return None

    lax.fori_loop(0, _N_CHUNKS // 2, pair_body, None)
    wait_writeout(1)


@jax.jit
def _embed(x1d, table):
    mesh = plsc.VectorSubcoreMesh(core_axis_name="c", subcore_axis_name="s")
    fn = functools.partial(
        pl.kernel,
        mesh=mesh,
        out_type=jax.ShapeDtypeStruct((BATCH, SEQ, D_MODEL), jnp.float32),
        scratch_types=[
            pltpu.VMEM((_B_PER_W,), jnp.int32),
            pltpu.VMEM((2, _CHUNK, D_MODEL), jnp.float32),
            pltpu.SemaphoreType.DMA,
            pltpu.SemaphoreType.DMA,
            pltpu.SemaphoreType.DMA,
            pltpu.SemaphoreType.DMA,
        ],
        compiler_params=pltpu.CompilerParams(use_tc_tiling_on_sc=False),
    )(_emb_kernel)
    return fn(x1d, table)


def kernel(x, table):
    x1d = x.reshape(_B).astype(jnp.int32)
    return _embed(x1d, table)
